# baseline (speedup 1.0000x reference)
import functools

import jax
import jax.numpy as jnp
from jax import lax
from jax.experimental import pallas as pl
from jax.experimental.pallas import tpu as pltpu
from jax.experimental.pallas import tpu_sc as plsc

N = 10000
E = 320000
D = 128
NC, NS = 2, 16
NW = NC * NS
K = 128
C0 = 96
C1 = 64
CPS0 = C0 // 2
TOTCH = NS * (C0 + C1)
EPAD = TOTCH * K
NP = 10240
RPT = NP // NS
BM = 1024


def _sc_mesh():
    return plsc.VectorSubcoreMesh(core_axis_name="c", subcore_axis_name="s")


@functools.partial(
    pl.kernel,
    out_type=jax.ShapeDtypeStruct((NC, NP, 16), jnp.float32),
    mesh=_sc_mesh(),
    scratch_types=[
        pltpu.VMEM_SHARED((NP, 16), jnp.float32),
        pltpu.VMEM((C0, K), jnp.int32),
        pltpu.VMEM((K, 16), jnp.float32),
        pltpu.VMEM((RPT, 16), jnp.float32),
        pltpu.SemaphoreType.DMA,
    ],
)
def _deg_kernel(to_hbm, degp_hbm, acc, to_v, ones_v, zero_v, dsem):
    cid = lax.axis_index("c")
    sid = lax.axis_index("s")

    def fill_zero(i, carry):
        zero_v[i] = jnp.zeros((16,), jnp.float32)
        return carry

    lax.fori_loop(0, RPT, fill_zero, 0)

    def fill_ones(i, carry):
        ones_v[i] = jnp.ones((16,), jnp.float32)
        return carry

    lax.fori_loop(0, K, fill_ones, 0)

    pltpu.sync_copy(zero_v, acc.at[pl.ds(sid * RPT, RPT)])
    plsc.subcore_barrier()

    DGRP = 8

    def run(cbase, nch):
        pltpu.sync_copy(to_hbm.at[pl.ds(cbase, nch)], to_v.at[pl.ds(0, nch)])

        def body(g, carry):
            for b in range(DGRP):
                pltpu.async_copy(ones_v, acc.at[to_v.at[g * DGRP + b]], dsem,
                                 add=True)
            for b in range(DGRP):
                pltpu.make_async_copy(ones_v, acc.at[to_v.at[0]], dsem).wait()
            return carry

        lax.fori_loop(0, nch // DGRP, body, 0)

    @pl.when(cid == 0)
    def _c0():
        run(sid * C0, C0)

    @pl.when(cid == 1)
    def _c1():
        run(NS * C0 + sid * C1, C1)

    plsc.subcore_barrier()

    sl = pl.ds(sid * RPT, RPT)
    pltpu.sync_copy(acc.at[sl], degp_hbm.at[cid, sl])


@functools.partial(
    pl.kernel,
    out_type=jax.ShapeDtypeStruct((NC, NP, D), jnp.float32),
    mesh=_sc_mesh(),
    scratch_types=[
        pltpu.VMEM_SHARED((NP, D), jnp.float32),
        pltpu.VMEM((CPS0, K), jnp.int32),
        pltpu.VMEM((CPS0, K), jnp.int32),
        pltpu.VMEM((K, D), jnp.float32),
        pltpu.SemaphoreType.DMA,
    ],
)
def _edge_kernel(g_hbm, from_hbm, to_hbm, parts_hbm, acc, from_v, to_v,
                 rows, gsem):
    cid = lax.axis_index("c")
    sid = lax.axis_index("s")

    def fill_zero(t, carry):
        rows[t // 8, pl.ds((t % 8) * 16, 16)] = jnp.zeros((16,), jnp.float32)
        return carry

    lax.fori_loop(0, K * 8, fill_zero, 0)
    for r in range(RPT // K):
        pltpu.sync_copy(rows, acc.at[pl.ds(sid * RPT + r * K, K)])
    plsc.subcore_barrier()

    def run_serial(cbase, cps):
        pltpu.sync_copy(from_hbm.at[pl.ds(cbase, cps)],
                        from_v.at[pl.ds(0, cps)])
        pltpu.sync_copy(to_hbm.at[pl.ds(cbase, cps)], to_v.at[pl.ds(0, cps)])

        def body(j, carry):
            pltpu.async_copy(g_hbm.at[from_v.at[j]], rows, gsem).wait()
            pltpu.sync_copy(rows, acc.at[to_v.at[j]], add=True)
            return carry

        lax.fori_loop(0, cps, body, 0)

    @pl.when(cid == 0)
    def _c0():
        for st in range(C0 // CPS0):
            run_serial(sid * C0 + st * CPS0, CPS0)

    @pl.when(cid == 1)
    def _c1():
        for st in range(2):
            run_serial(NS * C0 + sid * C1 + st * (C1 // 2), C1 // 2)

    plsc.subcore_barrier()

    sl = pl.ds(sid * RPT, RPT)
    pltpu.sync_copy(acc.at[sl], parts_hbm.at[cid, sl])


def _dinv(d0_ref, d1_ref):
    deg = d0_ref[...][:, 0:1] + d1_ref[...][:, 0:1] + 1.0
    return lax.rsqrt(deg)


def _mm_scale_body(x_ref, wt_ref, b_ref, d0_ref, d1_ref, g_ref):
    h = jnp.dot(x_ref[...], wt_ref[...], preferred_element_type=jnp.float32)
    g_ref[...] = (h + b_ref[...]) * _dinv(d0_ref, d1_ref)


def _combine_mm_body(p0_ref, p1_ref, g0_ref, d0_ref, d1_ref, wt_ref, b_ref, g1_ref):
    dinv = _dinv(d0_ref, d1_ref)
    s = p0_ref[...] + p1_ref[...] + g0_ref[...]
    o = jnp.maximum(s * dinv, 0.0)
    h = jnp.dot(o, wt_ref[...], preferred_element_type=jnp.float32)
    g1_ref[...] = (h + b_ref[...]) * dinv


def _final_body(p0_ref, p1_ref, g1_ref, d0_ref, d1_ref, out_ref):
    s = p0_ref[...] + p1_ref[...] + g1_ref[...]
    out_ref[...] = s * _dinv(d0_ref, d1_ref)


def _row_spec():
    return pl.BlockSpec((BM, D), lambda i: (i, 0))


def _deg_spec():
    return pl.BlockSpec((BM, 16), lambda i: (i, 0))


def _full_spec(shape):
    return pl.BlockSpec(shape, lambda i: (0,) * len(shape))


def _mm_scale(x_p, wt, br, d0, d1):
    return pl.pallas_call(
        _mm_scale_body,
        grid=(NP // BM,),
        in_specs=[_row_spec(), _full_spec((D, D)), _full_spec((1, D)),
                  _deg_spec(), _deg_spec()],
        out_specs=_row_spec(),
        out_shape=jax.ShapeDtypeStruct((NP, D), jnp.float32),
    )(x_p, wt, br, d0, d1)


def _combine_mm(p0, p1, g0, d0, d1, wt, br):
    return pl.pallas_call(
        _combine_mm_body,
        grid=(NP // BM,),
        in_specs=[_row_spec(), _row_spec(), _row_spec(), _deg_spec(),
                  _deg_spec(), _full_spec((D, D)), _full_spec((1, D))],
        out_specs=_row_spec(),
        out_shape=jax.ShapeDtypeStruct((NP, D), jnp.float32),
    )(p0, p1, g0, d0, d1, wt, br)


def _final(p0, p1, g1, d0, d1):
    return pl.pallas_call(
        _final_body,
        grid=(NP // BM,),
        in_specs=[_row_spec(), _row_spec(), _row_spec(), _deg_spec(),
                  _deg_spec()],
        out_specs=_row_spec(),
        out_shape=jax.ShapeDtypeStruct((NP, D), jnp.float32),
    )(p0, p1, g1, d0, d1)


def kernel(x, edge_index, W0, b0, W1, b1):
    from_p = jnp.concatenate(
        [edge_index[0], jnp.zeros((EPAD - E,), jnp.int32)])
    to_p = jnp.concatenate(
        [edge_index[1], jnp.full((EPAD - E,), N, jnp.int32)])
    from_h = from_p.reshape(TOTCH, K)
    to_h = to_p.reshape(TOTCH, K)
    x_p = jnp.pad(x, ((0, NP - N), (0, 0)))
    wt0 = W0.T
    wt1 = W1.T
    b0r = b0.reshape(1, D)
    b1r = b1.reshape(1, D)

    degp = _deg_kernel(to_h)
    d0, d1 = degp[0], degp[1]

    g0 = _mm_scale(x_p, wt0, b0r, d0, d1)
    parts0 = _edge_kernel(g0, from_h, to_h)
    g1 = _combine_mm(parts0[0], parts0[1], g0, d0, d1, wt1, b1r)
    parts1 = _edge_kernel(g1, from_h, to_h)
    out = _final(parts1[0], parts1[1], g1, d0, d1)
    return out[:N]


# speedup vs baseline: 2.1255x; 2.1255x over previous
import functools

import jax
import jax.numpy as jnp
from jax import lax
from jax.experimental import pallas as pl
from jax.experimental.pallas import tpu as pltpu
from jax.experimental.pallas import tpu_sc as plsc

N = 10000
E = 320000
D = 128
NC, NS = 2, 16
NW = NC * NS
K = 128
C0 = 80
C1 = 80
CPS0 = C0 // 2
TOTCH = NS * (C0 + C1)
EPAD = TOTCH * K
NP = 10240
RPT = NP // NS
BM = 1024


def _sc_mesh():
    return plsc.VectorSubcoreMesh(core_axis_name="c", subcore_axis_name="s")


@functools.partial(
    pl.kernel,
    out_type=jax.ShapeDtypeStruct((NC, NP, 16), jnp.float32),
    mesh=_sc_mesh(),
    scratch_types=[
        pltpu.VMEM_SHARED((NP, 16), jnp.float32),
        pltpu.VMEM((C0, K), jnp.int32),
        pltpu.VMEM((K, 16), jnp.float32),
        pltpu.VMEM((RPT, 16), jnp.float32),
        pltpu.SemaphoreType.DMA,
    ],
)
def _deg_kernel(to_hbm, degp_hbm, acc, to_v, ones_v, zero_v, dsem):
    cid = lax.axis_index("c")
    sid = lax.axis_index("s")

    def fill_zero(i, carry):
        zero_v[i] = jnp.zeros((16,), jnp.float32)
        return carry

    lax.fori_loop(0, RPT, fill_zero, 0)

    def fill_ones(i, carry):
        ones_v[i] = jnp.ones((16,), jnp.float32)
        return carry

    lax.fori_loop(0, K, fill_ones, 0)

    pltpu.sync_copy(zero_v, acc.at[pl.ds(sid * RPT, RPT)])
    plsc.subcore_barrier()

    DGRP = 8

    def run(cbase, nch):
        pltpu.sync_copy(to_hbm.at[pl.ds(cbase, nch)], to_v.at[pl.ds(0, nch)])

        def body(g, carry):
            for b in range(DGRP):
                pltpu.async_copy(ones_v, acc.at[to_v.at[g * DGRP + b]], dsem,
                                 add=True)
            for b in range(DGRP):
                pltpu.make_async_copy(ones_v, acc.at[to_v.at[0]], dsem).wait()
            return carry

        lax.fori_loop(0, nch // DGRP, body, 0)

    @pl.when(cid == 0)
    def _c0():
        run(sid * C0, C0)

    @pl.when(cid == 1)
    def _c1():
        run(NS * C0 + sid * C1, C1)

    plsc.subcore_barrier()

    sl = pl.ds(sid * RPT, RPT)
    pltpu.sync_copy(acc.at[sl], degp_hbm.at[cid, sl])


@functools.partial(
    pl.kernel,
    out_type=jax.ShapeDtypeStruct((NC, NP, D), jnp.float32),
    mesh=_sc_mesh(),
    scratch_types=[
        pltpu.VMEM_SHARED((NP, D), jnp.float32),
        pltpu.VMEM((CPS0, K), jnp.int32),
        pltpu.VMEM((CPS0, K), jnp.int32),
        pltpu.VMEM((K, D), jnp.float32),
        pltpu.SemaphoreType.DMA,
    ],
)
def _edge_kernel(g_hbm, from_hbm, to_hbm, parts_hbm, acc, from_v, to_v,
                 rows, gsem):
    cid = lax.axis_index("c")
    sid = lax.axis_index("s")

    def fill_zero(t, carry):
        rows[t // 8, pl.ds((t % 8) * 16, 16)] = jnp.zeros((16,), jnp.float32)
        return carry

    lax.fori_loop(0, K * 8, fill_zero, 0)
    for r in range(RPT // K):
        pltpu.sync_copy(rows, acc.at[pl.ds(sid * RPT + r * K, K)])
    plsc.subcore_barrier()

    def run_serial(cbase, cps):
        pltpu.sync_copy(from_hbm.at[pl.ds(cbase, cps)],
                        from_v.at[pl.ds(0, cps)])
        pltpu.sync_copy(to_hbm.at[pl.ds(cbase, cps)], to_v.at[pl.ds(0, cps)])

        def body(j, carry):
            pltpu.async_copy(g_hbm.at[from_v.at[j]], rows, gsem).wait()
            pltpu.sync_copy(rows, acc.at[to_v.at[j]], add=True)
            return carry

        lax.fori_loop(0, cps, body, 0)

    @pl.when(cid == 0)
    def _c0():
        for st in range(C0 // CPS0):
            run_serial(sid * C0 + st * CPS0, CPS0)

    @pl.when(cid == 1)
    def _c1():
        for st in range(2):
            run_serial(NS * C0 + sid * C1 + st * (C1 // 2), C1 // 2)

    plsc.subcore_barrier()

    sl = pl.ds(sid * RPT, RPT)
    pltpu.sync_copy(acc.at[sl], parts_hbm.at[cid, sl])


def _dinv(d0_ref, d1_ref):
    deg = d0_ref[...][:, 0:1] + d1_ref[...][:, 0:1] + 1.0
    return lax.rsqrt(deg)


def _mm_scale_body(x_ref, wt_ref, b_ref, d0_ref, d1_ref, g_ref):
    h = jnp.dot(x_ref[...], wt_ref[...], preferred_element_type=jnp.float32)
    g_ref[...] = (h + b_ref[...]) * _dinv(d0_ref, d1_ref)


def _combine_mm_body(p0_ref, p1_ref, g0_ref, d0_ref, d1_ref, wt_ref, b_ref, g1_ref):
    dinv = _dinv(d0_ref, d1_ref)
    s = p0_ref[...] + p1_ref[...] + g0_ref[...]
    o = jnp.maximum(s * dinv, 0.0)
    h = jnp.dot(o, wt_ref[...], preferred_element_type=jnp.float32)
    g1_ref[...] = (h + b_ref[...]) * dinv


def _final_body(p0_ref, p1_ref, g1_ref, d0_ref, d1_ref, out_ref):
    s = p0_ref[...] + p1_ref[...] + g1_ref[...]
    out_ref[...] = s * _dinv(d0_ref, d1_ref)


def _row_spec():
    return pl.BlockSpec((BM, D), lambda i: (i, 0))


def _deg_spec():
    return pl.BlockSpec((BM, 16), lambda i: (i, 0))


def _full_spec(shape):
    return pl.BlockSpec(shape, lambda i: (0,) * len(shape))


def _mm_scale(x_p, wt, br, d0, d1):
    return pl.pallas_call(
        _mm_scale_body,
        grid=(NP // BM,),
        in_specs=[_row_spec(), _full_spec((D, D)), _full_spec((1, D)),
                  _deg_spec(), _deg_spec()],
        out_specs=_row_spec(),
        out_shape=jax.ShapeDtypeStruct((NP, D), jnp.float32),
    )(x_p, wt, br, d0, d1)


def _combine_mm(p0, p1, g0, d0, d1, wt, br):
    return pl.pallas_call(
        _combine_mm_body,
        grid=(NP // BM,),
        in_specs=[_row_spec(), _row_spec(), _row_spec(), _deg_spec(),
                  _deg_spec(), _full_spec((D, D)), _full_spec((1, D))],
        out_specs=_row_spec(),
        out_shape=jax.ShapeDtypeStruct((NP, D), jnp.float32),
    )(p0, p1, g0, d0, d1, wt, br)


def _final(p0, p1, g1, d0, d1):
    return pl.pallas_call(
        _final_body,
        grid=(NP // BM,),
        in_specs=[_row_spec(), _row_spec(), _row_spec(), _deg_spec(),
                  _deg_spec()],
        out_specs=_row_spec(),
        out_shape=jax.ShapeDtypeStruct((NP, D), jnp.float32),
    )(p0, p1, g1, d0, d1)


def kernel(x, edge_index, W0, b0, W1, b1):
    npad = EPAD - E
    pad_from = (jnp.arange(npad, dtype=jnp.int32) * 97) % N
    pad_to = N + (jnp.arange(npad, dtype=jnp.int32) % (NP - N))
    from_p = jnp.concatenate([edge_index[0], pad_from])
    to_p = jnp.concatenate([edge_index[1], pad_to])
    CPW = TOTCH // NW
    from_h = from_p.reshape(CPW, NW, K).swapaxes(0, 1).reshape(TOTCH, K)
    to_h = to_p.reshape(CPW, NW, K).swapaxes(0, 1).reshape(TOTCH, K)
    x_p = jnp.pad(x, ((0, NP - N), (0, 0)))
    wt0 = W0.T
    wt1 = W1.T
    b0r = b0.reshape(1, D)
    b1r = b1.reshape(1, D)

    degp = _deg_kernel(to_h)
    d0, d1 = degp[0], degp[1]

    g0 = _mm_scale(x_p, wt0, b0r, d0, d1)
    parts0 = _edge_kernel(g0, from_h, to_h)
    g1 = _combine_mm(parts0[0], parts0[1], g0, d0, d1, wt1, b1r)
    parts1 = _edge_kernel(g1, from_h, to_h)
    out = _final(parts1[0], parts1[1], g1, d0, d1)
    return out[:N]


# speedup vs baseline: 2.3840x; 1.1216x over previous
import functools

import jax
import jax.numpy as jnp
from jax import lax
from jax.experimental import pallas as pl
from jax.experimental.pallas import tpu as pltpu
from jax.experimental.pallas import tpu_sc as plsc

N = 10000
E = 320000
D = 128
NC, NS = 2, 16
NW = NC * NS
K = 128
NBUF = 2
C0 = 80
C1 = 80
CPS0 = C0 // 2
TOTCH = NS * (C0 + C1)
EPAD = TOTCH * K
NP = 10240
RPT = NP // NS
BM = 1024


def _sc_mesh():
    return plsc.VectorSubcoreMesh(core_axis_name="c", subcore_axis_name="s")


@functools.partial(
    pl.kernel,
    out_type=jax.ShapeDtypeStruct((NC, NP, 16), jnp.float32),
    mesh=_sc_mesh(),
    scratch_types=[
        pltpu.VMEM_SHARED((NP, 16), jnp.float32),
        pltpu.VMEM((C0, K), jnp.int32),
        pltpu.VMEM((K, 16), jnp.float32),
        pltpu.VMEM((RPT, 16), jnp.float32),
        pltpu.SemaphoreType.DMA,
    ],
)
def _deg_kernel(to_hbm, degp_hbm, acc, to_v, ones_v, zero_v, dsem):
    cid = lax.axis_index("c")
    sid = lax.axis_index("s")

    def fill_zero(i, carry):
        zero_v[i] = jnp.zeros((16,), jnp.float32)
        return carry

    lax.fori_loop(0, RPT, fill_zero, 0)

    def fill_ones(i, carry):
        ones_v[i] = jnp.ones((16,), jnp.float32)
        return carry

    lax.fori_loop(0, K, fill_ones, 0)

    pltpu.sync_copy(zero_v, acc.at[pl.ds(sid * RPT, RPT)])
    plsc.subcore_barrier()

    DGRP = 8

    def run(cbase, nch):
        pltpu.sync_copy(to_hbm.at[pl.ds(cbase, nch)], to_v.at[pl.ds(0, nch)])

        def body(g, carry):
            for b in range(DGRP):
                pltpu.async_copy(ones_v, acc.at[to_v.at[g * DGRP + b]], dsem,
                                 add=True)
            for b in range(DGRP):
                pltpu.make_async_copy(ones_v, acc.at[to_v.at[0]], dsem).wait()
            return carry

        lax.fori_loop(0, nch // DGRP, body, 0)

    @pl.when(cid == 0)
    def _c0():
        run(sid * C0, C0)

    @pl.when(cid == 1)
    def _c1():
        run(NS * C0 + sid * C1, C1)

    plsc.subcore_barrier()

    sl = pl.ds(sid * RPT, RPT)
    pltpu.sync_copy(acc.at[sl], degp_hbm.at[cid, sl])


@functools.partial(
    pl.kernel,
    out_type=jax.ShapeDtypeStruct((NC, NP, D), jnp.float32),
    mesh=_sc_mesh(),
    scratch_types=[
        pltpu.VMEM_SHARED((NP, D), jnp.float32),
        pltpu.VMEM((CPS0, K), jnp.int32),
        pltpu.VMEM((CPS0, K), jnp.int32),
        [pltpu.VMEM((K, D), jnp.float32)] * NBUF,
        [pltpu.SemaphoreType.DMA] * NBUF,
        [pltpu.SemaphoreType.DMA] * NBUF,
    ],
)
def _edge_kernel(g_hbm, from_hbm, to_hbm, parts_hbm, acc, from_v, to_v,
                 rows, gsem, ssem):
    cid = lax.axis_index("c")
    sid = lax.axis_index("s")

    def fill_zero(t, carry):
        rows[0][t // 8, pl.ds((t % 8) * 16, 16)] = jnp.zeros((16,), jnp.float32)
        return carry

    lax.fori_loop(0, K * 8, fill_zero, 0)
    for r in range(RPT // K):
        pltpu.sync_copy(rows[0], acc.at[pl.ds(sid * RPT + r * K, K)])
    plsc.subcore_barrier()

    def run_stage(cbase, cps):
        pltpu.sync_copy(from_hbm.at[pl.ds(cbase, cps)],
                        from_v.at[pl.ds(0, cps)])
        pltpu.sync_copy(to_hbm.at[pl.ds(cbase, cps)], to_v.at[pl.ds(0, cps)])

        def body(g, carry):
            for b in range(NBUF):
                pltpu.async_copy(g_hbm.at[from_v.at[g * NBUF + b]], rows[b],
                                 gsem[b])
            for b in range(NBUF):
                pltpu.make_async_copy(g_hbm.at[from_v.at[g * NBUF + b]],
                                      rows[b], gsem[b]).wait()
            for b in range(NBUF):
                pltpu.async_copy(rows[b], acc.at[to_v.at[g * NBUF + b]],
                                 ssem[b], add=True)
            for b in range(NBUF):
                pltpu.make_async_copy(rows[b], acc.at[to_v.at[g * NBUF + b]],
                                      ssem[b]).wait()
            return carry

        lax.fori_loop(0, cps // NBUF, body, 0)

    @pl.when(cid == 0)
    def _c0():
        for st in range(C0 // CPS0):
            run_stage(sid * C0 + st * CPS0, CPS0)

    @pl.when(cid == 1)
    def _c1():
        for st in range(2):
            run_stage(NS * C0 + sid * C1 + st * (C1 // 2), C1 // 2)

    plsc.subcore_barrier()

    sl = pl.ds(sid * RPT, RPT)
    pltpu.sync_copy(acc.at[sl], parts_hbm.at[cid, sl])


def _dinv(d0_ref, d1_ref):
    deg = d0_ref[...][:, 0:1] + d1_ref[...][:, 0:1] + 1.0
    return lax.rsqrt(deg)


def _mm_scale_body(x_ref, wt_ref, b_ref, d0_ref, d1_ref, g_ref):
    h = jnp.dot(x_ref[...], wt_ref[...], preferred_element_type=jnp.float32)
    g_ref[...] = (h + b_ref[...]) * _dinv(d0_ref, d1_ref)


def _combine_mm_body(p0_ref, p1_ref, g0_ref, d0_ref, d1_ref, wt_ref, b_ref, g1_ref):
    dinv = _dinv(d0_ref, d1_ref)
    s = p0_ref[...] + p1_ref[...] + g0_ref[...]
    o = jnp.maximum(s * dinv, 0.0)
    h = jnp.dot(o, wt_ref[...], preferred_element_type=jnp.float32)
    g1_ref[...] = (h + b_ref[...]) * dinv


def _final_body(p0_ref, p1_ref, g1_ref, d0_ref, d1_ref, out_ref):
    s = p0_ref[...] + p1_ref[...] + g1_ref[...]
    out_ref[...] = s * _dinv(d0_ref, d1_ref)


def _row_spec():
    return pl.BlockSpec((BM, D), lambda i: (i, 0))


def _deg_spec():
    return pl.BlockSpec((BM, 16), lambda i: (i, 0))


def _full_spec(shape):
    return pl.BlockSpec(shape, lambda i: (0,) * len(shape))


def _mm_scale(x_p, wt, br, d0, d1):
    return pl.pallas_call(
        _mm_scale_body,
        grid=(NP // BM,),
        in_specs=[_row_spec(), _full_spec((D, D)), _full_spec((1, D)),
                  _deg_spec(), _deg_spec()],
        out_specs=_row_spec(),
        out_shape=jax.ShapeDtypeStruct((NP, D), jnp.float32),
    )(x_p, wt, br, d0, d1)


def _combine_mm(p0, p1, g0, d0, d1, wt, br):
    return pl.pallas_call(
        _combine_mm_body,
        grid=(NP // BM,),
        in_specs=[_row_spec(), _row_spec(), _row_spec(), _deg_spec(),
                  _deg_spec(), _full_spec((D, D)), _full_spec((1, D))],
        out_specs=_row_spec(),
        out_shape=jax.ShapeDtypeStruct((NP, D), jnp.float32),
    )(p0, p1, g0, d0, d1, wt, br)


def _final(p0, p1, g1, d0, d1):
    return pl.pallas_call(
        _final_body,
        grid=(NP // BM,),
        in_specs=[_row_spec(), _row_spec(), _row_spec(), _deg_spec(),
                  _deg_spec()],
        out_specs=_row_spec(),
        out_shape=jax.ShapeDtypeStruct((NP, D), jnp.float32),
    )(p0, p1, g1, d0, d1)


def kernel(x, edge_index, W0, b0, W1, b1):
    npad = EPAD - E
    pad_from = (jnp.arange(npad, dtype=jnp.int32) * 97) % N
    pad_to = N + (jnp.arange(npad, dtype=jnp.int32) % (NP - N))
    from_p = jnp.concatenate([edge_index[0], pad_from])
    to_p = jnp.concatenate([edge_index[1], pad_to])
    CPW = TOTCH // NW
    from_h = from_p.reshape(CPW, NW, K).swapaxes(0, 1).reshape(TOTCH, K)
    to_h = to_p.reshape(CPW, NW, K).swapaxes(0, 1).reshape(TOTCH, K)
    x_p = jnp.pad(x, ((0, NP - N), (0, 0)))
    wt0 = W0.T
    wt1 = W1.T
    b0r = b0.reshape(1, D)
    b1r = b1.reshape(1, D)

    degp = _deg_kernel(to_h)
    d0, d1 = degp[0], degp[1]

    g0 = _mm_scale(x_p, wt0, b0r, d0, d1)
    parts0 = _edge_kernel(g0, from_h, to_h)
    g1 = _combine_mm(parts0[0], parts0[1], g0, d0, d1, wt1, b1r)
    parts1 = _edge_kernel(g1, from_h, to_h)
    out = _final(parts1[0], parts1[1], g1, d0, d1)
    return out[:N]
